# baseline (speedup 1.0000x reference)
import functools

import jax
import jax.numpy as jnp
import numpy as np
from jax import lax
from jax.experimental import pallas as pl
from jax.experimental.pallas import tpu as pltpu
from jax.experimental.pallas import tpu_sc as plsc

_N = 10000
_E = 320000
_DH = 128
_HD = 64
_NC = 2
_NS = 16
_NW = _NC * _NS
_CH = 125
_NCHA = (_E // _NS) // _CH
_NCHH = _NCHA // 2
_GRP = 4
_RPT = _N // _NS
_RZ = 125
_NZ = _RPT // _RZ
_DEGW = 16
_BN = 1000


def _sc_mesh():
    return plsc.VectorSubcoreMesh(core_axis_name="c", subcore_axis_name="s")


def _fast_rsqrt(v):
    i = plsc.bitcast(v, jnp.int32)
    i = jnp.int32(0x5F3759DF) - (i >> 1)
    y = plsc.bitcast(i, jnp.float32)
    for _ in range(3):
        y = y * (1.5 - 0.5 * v * y * y)
    return y


def _agg_pipeline(tab, src_v, dst_v, acc, bufs, gsem, ssem, nch):
    for b in range(_GRP):
        pltpu.async_copy(tab.at[src_v.at[b]], bufs[b], gsem[b])

    def grp(i, carry):
        for b in range(_GRP):
            c = i * _GRP + b
            pltpu.make_async_copy(
                tab.at[src_v.at[c]], bufs[b], gsem[b]).wait()
            pltpu.async_copy(
                bufs[b], acc.at[dst_v.at[c]], ssem[b], add=True)
        for b in range(_GRP):
            c = i * _GRP + b
            pltpu.make_async_copy(
                bufs[b], acc.at[dst_v.at[c]], ssem[b]).wait()
            pltpu.async_copy(
                tab.at[src_v.at[c + _GRP]], bufs[b], gsem[b])
        return carry

    lax.fori_loop(0, nch // _GRP - 1, grp, 0)
    base = nch - _GRP
    for b in range(_GRP):
        pltpu.make_async_copy(
            tab.at[src_v.at[base + b]], bufs[b], gsem[b]).wait()
        pltpu.async_copy(
            bufs[b], acc.at[dst_v.at[base + b]], ssem[b], add=True)
    for b in range(_GRP):
        pltpu.make_async_copy(
            bufs[b], acc.at[dst_v.at[base + b]], ssem[b]).wait()


@functools.cache
def _build_mega():
    scratch = [
        pltpu.VMEM((_NCHH, _CH), jnp.int32),
        pltpu.VMEM((_NCHH, _CH), jnp.int32),
    ]
    scratch += [pltpu.VMEM((_CH, _HD), jnp.float32) for _ in range(_GRP)]
    scratch += [
        pltpu.VMEM((_CH, _DEGW), jnp.float32),
        pltpu.VMEM((_CH, _DEGW), jnp.float32),
        pltpu.VMEM((_CH, _DEGW), jnp.float32),
        pltpu.VMEM((_CH, _HD), jnp.float32),
        pltpu.VMEM_SHARED((_N, _HD), jnp.float32),
        pltpu.VMEM_SHARED((_N, _DEGW), jnp.float32),
    ]
    scratch += [pltpu.SemaphoreType.DMA for _ in range(2 * _GRP + 1)]

    @functools.partial(
        pl.kernel,
        out_type=[
            jax.ShapeDtypeStruct((_NC, 3, _NS, _RPT, _HD), jnp.float32),
            jax.ShapeDtypeStruct((_NC, 3, _NS, _RPT, _DEGW), jnp.float32),
            jax.ShapeDtypeStruct((3, 2 * _N, _HD), jnp.float32),
        ],
        mesh=_sc_mesh(),
        scratch_types=tuple(scratch),
        compiler_params=pltpu.CompilerParams(
            use_tc_tiling_on_sc=False, needs_layout_passes=False),
    )
    def _mega(src_hbm, dst_hbm, xst_hbm, ones_hbm, zeros16_hbm, zeros64_hbm,
              agg_out, dis_out, xs_tab,
              src_v, dst_v, b0, b1, b2, b3, ones_v, hb, db, xb,
              acc_f, acc_d, g0, g1, g2, g3, s0, s1, s2, s3, dsem):
        bufs = (b0, b1, b2, b3)
        gsem = (g0, g1, g2, g3)
        ssem = (s0, s1, s2, s3)
        cid = lax.axis_index("c")
        sid = lax.axis_index("s")
        base = sid * _RPT
        pltpu.sync_copy(ones_hbm, ones_v)
        for k in range(3):
            pltpu.sync_copy(zeros16_hbm, hb)
            pltpu.sync_copy(zeros64_hbm, bufs[0])
            for z in range(_NZ):
                pltpu.sync_copy(hb, acc_d.at[pl.ds(base + z * _RZ, _RZ)])
                pltpu.sync_copy(bufs[0], acc_f.at[pl.ds(base + z * _RZ, _RZ)])
            plsc.subcore_barrier()

            for h in range(2):
                pltpu.sync_copy(dst_hbm.at[k, sid, pl.ds(h * _NCHH, _NCHH)],
                                dst_v)

                def fire(c, carry):
                    pltpu.async_copy(
                        ones_v, acc_d.at[dst_v.at[c]], dsem, add=True)
                    return carry

                lax.fori_loop(0, _NCHH, fire, 0)

                def drain(c, carry):
                    pltpu.make_async_copy(
                        ones_v, acc_d.at[dst_v.at[c]], dsem).wait()
                    return carry

                lax.fori_loop(0, _NCHH, drain, 0)
            plsc.subcore_barrier()

            for z in range(_NZ):
                pltpu.sync_copy(acc_d.at[pl.ds(base + z * _RZ, _RZ)], hb)
                pltpu.sync_copy(
                    xst_hbm.at[pl.ds(cid * _N + base + z * _RZ, _RZ)], xb)

                def row(r, carry):
                    y = _fast_rsqrt(hb[r, :] + 1.0)
                    db[r, :] = y
                    for q in range(_HD // 16):
                        xb[r, pl.ds(q * 16, 16)] = xb[r, pl.ds(q * 16, 16)] * y
                    return carry

                lax.fori_loop(0, _RZ, row, 0)
                pltpu.sync_copy(
                    db, dis_out.at[cid, k, sid, pl.ds(z * _RZ, _RZ)])
                pltpu.sync_copy(
                    xb, xs_tab.at[k, pl.ds(cid * _N + base + z * _RZ, _RZ)])
            plsc.subcore_barrier()

            for h in range(2):
                pltpu.sync_copy(
                    src_hbm.at[k, cid, sid, pl.ds(h * _NCHH, _NCHH)], src_v)
                pltpu.sync_copy(
                    dst_hbm.at[k, sid, pl.ds(h * _NCHH, _NCHH)], dst_v)
                _agg_pipeline(xs_tab.at[k], src_v, dst_v, acc_f, bufs, gsem,
                              ssem, _NCHH)
            plsc.subcore_barrier()
            for z in range(_NZ):
                pltpu.sync_copy(acc_f.at[pl.ds(base + z * _RZ, _RZ)], bufs[0])
                pltpu.sync_copy(
                    bufs[0], agg_out.at[cid, k, sid, pl.ds(z * _RZ, _RZ)])

    return _mega


@functools.cache
def _build_agg1():
    scratch = [
        pltpu.VMEM((_NCHA, _CH), jnp.int32),
        pltpu.VMEM((_NCHA, _CH), jnp.int32),
    ]
    scratch += [pltpu.VMEM((_CH, _HD), jnp.float32) for _ in range(_GRP)]
    scratch += [pltpu.VMEM_SHARED((_N, _HD), jnp.float32)]
    scratch += [pltpu.SemaphoreType.DMA for _ in range(2 * _GRP)]

    @functools.partial(
        pl.kernel,
        out_type=jax.ShapeDtypeStruct((_NC, _NS, _RPT, _HD), jnp.float32),
        mesh=_sc_mesh(),
        scratch_types=tuple(scratch),
        compiler_params=pltpu.CompilerParams(use_tc_tiling_on_sc=False),
    )
    def _agg1(src_hbm, dst_hbm, zeros64_hbm, tab_hbm, out_hbm,
              src_v, dst_v, b0, b1, b2, b3, acc,
              g0, g1, g2, g3, s0, s1, s2, s3):
        bufs = (b0, b1, b2, b3)
        gsem = (g0, g1, g2, g3)
        ssem = (s0, s1, s2, s3)
        cid = lax.axis_index("c")
        sid = lax.axis_index("s")
        base = sid * _RPT
        pltpu.sync_copy(zeros64_hbm, bufs[0])
        for z in range(_NZ):
            pltpu.sync_copy(bufs[0], acc.at[pl.ds(base + z * _RZ, _RZ)])
        pltpu.sync_copy(src_hbm.at[cid, sid], src_v)
        pltpu.sync_copy(dst_hbm.at[0, sid], dst_v)
        plsc.subcore_barrier()
        _agg_pipeline(tab_hbm, src_v, dst_v, acc, bufs, gsem, ssem, _NCHA)
        plsc.subcore_barrier()
        for z in range(_NZ):
            pltpu.sync_copy(acc.at[pl.ds(base + z * _RZ, _RZ)], bufs[0])
            pltpu.sync_copy(bufs[0], out_hbm.at[cid, sid, pl.ds(z * _RZ, _RZ)])

    return _agg1


def _dot(a, b):
    return jnp.dot(a, b, preferred_element_type=jnp.float32,
                   precision=lax.Precision.HIGHEST)


def _stage2_body(dis_ref, agg_ref, xst_ref, w_ref, a_ref, c_ref,
                 e1st_ref, z01_ref):
    dis = dis_ref[...]
    u = [dis[:, k:k + 1] * jnp.concatenate(
            [agg_ref[0, k] + xst_ref[k, 0], agg_ref[1, k] + xst_ref[k, 1]],
            axis=1)
         for k in range(3)]
    e = jnp.maximum(_dot(u[0], w_ref[0]) * a_ref[0] + c_ref[0], 0.0)
    e1s = dis[:, 0:1] * e
    e1st_ref[0] = e1s[:, :_HD]
    e1st_ref[1] = e1s[:, _HD:]
    z0 = jnp.maximum(_dot(u[1], w_ref[1]) * a_ref[1] + c_ref[1], 0.0)
    z1 = jnp.maximum(_dot(u[2], w_ref[2]) * a_ref[2] + c_ref[2], 0.0)
    z01_ref[...] = jnp.concatenate([z0, z1], axis=1)


def _stage2(dis, agg, xst, wstk, avec, cvec):
    nb = _N // _BN
    return pl.pallas_call(
        _stage2_body,
        grid=(nb,),
        in_specs=[
            pl.BlockSpec((_BN, 3), lambda i: (i, 0)),
            pl.BlockSpec((2, 3, _BN, _HD), lambda i: (0, 0, i, 0)),
            pl.BlockSpec((3, 2, _BN, _HD), lambda i: (0, 0, i, 0)),
            pl.BlockSpec((3, _DH, _DH), lambda i: (0, 0, 0)),
            pl.BlockSpec((3, _DH), lambda i: (0, 0)),
            pl.BlockSpec((3, _DH), lambda i: (0, 0)),
        ],
        out_specs=[
            pl.BlockSpec((2, _BN, _HD), lambda i: (0, i, 0)),
            pl.BlockSpec((_BN, 2 * _DH), lambda i: (i, 0)),
        ],
        out_shape=[
            jax.ShapeDtypeStruct((2, _N, _HD), jnp.float32),
            jax.ShapeDtypeStruct((_N, 2 * _DH), jnp.float32),
        ],
    )(dis, agg, xst, wstk, avec, cvec)


def _stage4_body(dis_ref, a3_ref, e1st_ref, z01_ref, w2_ref, a2_ref, c2_ref,
                 wc_ref, bc_ref, out_ref):
    dis0 = dis_ref[:, 0:1]
    u = dis0 * jnp.concatenate(
        [a3_ref[0] + e1st_ref[0], a3_ref[1] + e1st_ref[1]], axis=1)
    e2 = jnp.maximum(_dot(u, w2_ref[...]) * a2_ref[0] + c2_ref[0], 0.0)
    h = jnp.concatenate([z01_ref[...], e2], axis=1)
    logits = _dot(h, wc_ref[...]) + bc_ref[0]
    m = jnp.max(logits, axis=-1, keepdims=True)
    ls = logits - m
    out_ref[...] = ls - jnp.log(jnp.sum(jnp.exp(ls), axis=-1, keepdims=True))


def _stage4(dis, a3, e1st, z01, w2, a2, c2, wc, bc):
    nb = _N // _BN
    return pl.pallas_call(
        _stage4_body,
        grid=(nb,),
        in_specs=[
            pl.BlockSpec((_BN, 3), lambda i: (i, 0)),
            pl.BlockSpec((2, _BN, _HD), lambda i: (0, i, 0)),
            pl.BlockSpec((2, _BN, _HD), lambda i: (0, i, 0)),
            pl.BlockSpec((_BN, 2 * _DH), lambda i: (i, 0)),
            pl.BlockSpec((_DH, _DH), lambda i: (0, 0)),
            pl.BlockSpec((1, _DH), lambda i: (0, 0)),
            pl.BlockSpec((1, _DH), lambda i: (0, 0)),
            pl.BlockSpec((3 * _DH, 64), lambda i: (0, 0)),
            pl.BlockSpec((1, 64), lambda i: (0, 0)),
        ],
        out_specs=pl.BlockSpec((_BN, 64), lambda i: (i, 0)),
        out_shape=jax.ShapeDtypeStruct((_N, 64), jnp.float32),
    )(dis, a3, e1st, z01, w2, a2, c2, wc, bc)


def kernel(x, adj, W_extra, b_extra, g_e, be_e, W_extra2, b_extra2, g_e2,
           be_e2, W0, b0, W1, b1, att, Wc, bc):
    f32 = jnp.float32
    src = adj[:, 0]
    dst = adj[:, 1]
    srcidx = (src.reshape(3, 1, _NS, _NCHA, _CH)
              + jnp.arange(_NC, dtype=adj.dtype).reshape(1, _NC, 1, 1, 1) * _N)
    dstidx = dst.reshape(3, _NS, _NCHA, _CH)
    xst = jnp.concatenate([x[:, :_HD], x[:, _HD:]], axis=0)
    ones16 = jnp.ones((_CH, _DEGW), f32)
    zeros16 = jnp.zeros((_CH, _DEGW), f32)
    zeros64 = jnp.zeros((_CH, _HD), f32)

    agg, dis_out, xs_tab = _build_mega()(
        srcidx, dstidx, xst, ones16, zeros16, zeros64)
    agg = agg.reshape(_NC, 3, _N, _HD)
    dis = dis_out[0, :, :, :, 0].reshape(3, _N).transpose(1, 0)

    mask = jax.nn.softmax(att, axis=0)
    rs = 1.0 / float(np.sqrt(1.0 + 1e-5))
    scl = jnp.stack([jnp.float32(1.0), mask[0], mask[1]])
    avec = jnp.stack([g_e * rs, jnp.ones_like(b0), jnp.ones_like(b1)]) * scl[:, None]
    cvec = jnp.stack([g_e * b_extra * rs + be_e, b0, b1]) * scl[:, None]
    wstk = jnp.stack([W_extra, W0, W1])

    e1st, z01 = _stage2(dis, agg, xs_tab.reshape(3, 2, _N, _HD),
                        wstk, avec, cvec)

    a3 = _build_agg1()(srcidx[0], dstidx, zeros64, e1st.reshape(2 * _N, _HD))
    a3 = a3.reshape(_NC, _N, _HD)

    a2 = (g_e2 * rs * mask[2]).reshape(1, _DH)
    c2 = ((g_e2 * b_extra2 * rs + be_e2) * mask[2]).reshape(1, _DH)
    z = _stage4(dis, a3, e1st, z01, W_extra2, a2, c2, Wc, bc.reshape(1, 64))
    return (z, jnp.zeros((), f32))


# speedup vs baseline: 1.1255x; 1.1255x over previous
import functools

import jax
import jax.numpy as jnp
import numpy as np
from jax import lax
from jax.experimental import pallas as pl
from jax.experimental.pallas import tpu as pltpu
from jax.experimental.pallas import tpu_sc as plsc

_N = 10000
_E = 320000
_DH = 128
_HD = 64
_NC = 2
_NS = 16
_NW = _NC * _NS
_CH = 125
_NCHD = (_E // _NW) // _CH
_NCHA = (_E // _NS) // _CH
_GRP = 5
_RPT = _N // _NS
_RZ = 125
_DEGW = 16
_BN = 1000


def _sc_mesh():
    return plsc.VectorSubcoreMesh(core_axis_name="c", subcore_axis_name="s")


@functools.cache
def _build_deg_kernel():
    return functools.partial(
        pl.kernel,
        out_type=jax.ShapeDtypeStruct((_NC, 3, _NS, _RPT, _DEGW), jnp.float32),
        mesh=_sc_mesh(),
        scratch_types=[
            pltpu.VMEM((_NCHD, _CH), jnp.int32),
            pltpu.VMEM((_CH, _DEGW), jnp.float32),
            pltpu.VMEM((_RPT, _DEGW), jnp.float32),
            pltpu.VMEM((_RZ, _DEGW), jnp.float32),
            pltpu.VMEM_SHARED((_N, _DEGW), jnp.float32),
            pltpu.SemaphoreType.DMA,
        ],
        compiler_params=pltpu.CompilerParams(use_tc_tiling_on_sc=False),
    )(_deg_body)


def _deg_body(dst_hbm, ones_hbm, zeros_hbm, out_hbm, dst_v, ones_v, zv, st, acc,
              dsem):
    cid = lax.axis_index("c")
    sid = lax.axis_index("s")
    wid = sid * _NC + cid
    pltpu.sync_copy(zeros_hbm, zv)
    pltpu.sync_copy(ones_hbm, ones_v)
    for k in range(3):
        pltpu.sync_copy(zv, acc.at[pl.ds(sid * _RPT, _RPT)])
        pltpu.sync_copy(dst_hbm.at[k, wid], dst_v)
        plsc.subcore_barrier()

        def fire(c, carry):
            pltpu.async_copy(ones_v, acc.at[dst_v.at[c]], dsem, add=True)
            return carry

        lax.fori_loop(0, _NCHD, fire, 0)

        def drain(c, carry):
            pltpu.make_async_copy(ones_v, acc.at[dst_v.at[c]], dsem).wait()
            return carry

        lax.fori_loop(0, _NCHD, drain, 0)
        plsc.subcore_barrier()
        for z in range(_RPT // _RZ):
            pltpu.sync_copy(acc.at[pl.ds(sid * _RPT + z * _RZ, _RZ)], st)
            pltpu.sync_copy(st, out_hbm.at[cid, k, sid, pl.ds(z * _RZ, _RZ)])


@functools.cache
def _make_agg(n):
    scratch = [
        pltpu.VMEM((_NCHA, _CH), jnp.int32),
        pltpu.VMEM((_NCHA, _CH), jnp.int32),
    ]
    scratch += [pltpu.VMEM((_CH, _HD), jnp.float32) for _ in range(_GRP)]
    scratch += [pltpu.VMEM_SHARED((_N, _HD), jnp.float32)]
    scratch += [pltpu.SemaphoreType.DMA for _ in range(2 * _GRP)]

    @functools.partial(
        pl.kernel,
        out_type=jax.ShapeDtypeStruct((_NC, n, _NS, _RPT, _HD), jnp.float32),
        mesh=_sc_mesh(),
        scratch_types=tuple(scratch),
        compiler_params=pltpu.CompilerParams(use_tc_tiling_on_sc=False),
    )
    def _agg(src_hbm, dst_hbm, zeros_hbm, *refs):
        tabs = refs[:n]
        out_hbm = refs[n]
        src_v, dst_v = refs[n + 1], refs[n + 2]
        bufs = refs[n + 3:n + 3 + _GRP]
        acc = refs[n + 3 + _GRP]
        gsem = refs[n + 4 + _GRP:n + 4 + 2 * _GRP]
        ssem = refs[n + 4 + 2 * _GRP:]
        cid = lax.axis_index("c")
        sid = lax.axis_index("s")
        for j in range(n):
            pltpu.sync_copy(zeros_hbm, bufs[0])
            for z in range(_RPT // _RZ):
                pltpu.sync_copy(bufs[0], acc.at[pl.ds(sid * _RPT + z * _RZ, _RZ)])
            pltpu.sync_copy(src_hbm.at[j, cid, sid], src_v)
            pltpu.sync_copy(dst_hbm.at[j, sid], dst_v)
            plsc.subcore_barrier()
            tab = tabs[j]

            for b in range(_GRP):
                pltpu.async_copy(tab.at[src_v.at[b]], bufs[b], gsem[b])

            def grp(i, carry, tab=tab):
                for b in range(_GRP):
                    c = i * _GRP + b
                    pltpu.make_async_copy(
                        tab.at[src_v.at[c]], bufs[b], gsem[b]).wait()
                    pltpu.async_copy(
                        bufs[b], acc.at[dst_v.at[c]], ssem[b], add=True)
                for b in range(_GRP):
                    c = i * _GRP + b
                    pltpu.make_async_copy(
                        bufs[b], acc.at[dst_v.at[c]], ssem[b]).wait()
                    pltpu.async_copy(
                        tab.at[src_v.at[c + _GRP]], bufs[b], gsem[b])
                return carry

            lax.fori_loop(0, _NCHA // _GRP - 1, grp, 0)
            base = _NCHA - _GRP
            for b in range(_GRP):
                pltpu.make_async_copy(
                    tab.at[src_v.at[base + b]], bufs[b], gsem[b]).wait()
                pltpu.async_copy(
                    bufs[b], acc.at[dst_v.at[base + b]], ssem[b], add=True)
            for b in range(_GRP):
                pltpu.make_async_copy(
                    bufs[b], acc.at[dst_v.at[base + b]], ssem[b]).wait()
            plsc.subcore_barrier()
            for z in range(_RPT // _RZ):
                pltpu.sync_copy(acc.at[pl.ds(sid * _RPT + z * _RZ, _RZ)], bufs[0])
                pltpu.sync_copy(bufs[0], out_hbm.at[cid, j, sid, pl.ds(z * _RZ, _RZ)])

    return _agg


def _prep_body(dp_ref, x_ref, xst_ref, dis_ref):
    dis = lax.rsqrt(dp_ref[:, :3] + dp_ref[:, 3:] + 1.0)
    dis_ref[...] = dis
    for k in range(3):
        sx = dis[:, k:k + 1] * x_ref[...]
        xst_ref[k, 0] = sx[:, :_HD]
        xst_ref[k, 1] = sx[:, _HD:]


def _prep(dp2, x):
    nb = _N // _BN
    return pl.pallas_call(
        _prep_body,
        grid=(nb,),
        in_specs=[
            pl.BlockSpec((_BN, 6), lambda i: (i, 0)),
            pl.BlockSpec((_BN, _DH), lambda i: (i, 0)),
        ],
        out_specs=[
            pl.BlockSpec((3, 2, _BN, _HD), lambda i: (0, 0, i, 0)),
            pl.BlockSpec((_BN, 3), lambda i: (i, 0)),
        ],
        out_shape=[
            jax.ShapeDtypeStruct((3, 2, _N, _HD), jnp.float32),
            jax.ShapeDtypeStruct((_N, 3), jnp.float32),
        ],
    )(dp2, x)


def _dot(a, b):
    return jnp.dot(a, b, preferred_element_type=jnp.float32,
                   precision=lax.Precision.HIGHEST)


def _stage2_body(dis_ref, agg_ref, xst_ref, w_ref, a_ref, c_ref,
                 e1st_ref, z01_ref):
    dis = dis_ref[...]
    u = [dis[:, k:k + 1] * jnp.concatenate(
            [agg_ref[0, k] + xst_ref[k, 0], agg_ref[1, k] + xst_ref[k, 1]],
            axis=1)
         for k in range(3)]
    e = jnp.maximum(_dot(u[0], w_ref[0]) * a_ref[0] + c_ref[0], 0.0)
    e1s = dis[:, 0:1] * e
    e1st_ref[0] = e1s[:, :_HD]
    e1st_ref[1] = e1s[:, _HD:]
    z0 = jnp.maximum(_dot(u[1], w_ref[1]) * a_ref[1] + c_ref[1], 0.0)
    z1 = jnp.maximum(_dot(u[2], w_ref[2]) * a_ref[2] + c_ref[2], 0.0)
    z01_ref[...] = jnp.concatenate([z0, z1], axis=1)


def _stage2(dis, agg, xst, wstk, avec, cvec):
    nb = _N // _BN
    return pl.pallas_call(
        _stage2_body,
        grid=(nb,),
        in_specs=[
            pl.BlockSpec((_BN, 3), lambda i: (i, 0)),
            pl.BlockSpec((2, 3, _BN, _HD), lambda i: (0, 0, i, 0)),
            pl.BlockSpec((3, 2, _BN, _HD), lambda i: (0, 0, i, 0)),
            pl.BlockSpec((3, _DH, _DH), lambda i: (0, 0, 0)),
            pl.BlockSpec((3, _DH), lambda i: (0, 0)),
            pl.BlockSpec((3, _DH), lambda i: (0, 0)),
        ],
        out_specs=[
            pl.BlockSpec((2, _BN, _HD), lambda i: (0, i, 0)),
            pl.BlockSpec((_BN, 2 * _DH), lambda i: (i, 0)),
        ],
        out_shape=[
            jax.ShapeDtypeStruct((2, _N, _HD), jnp.float32),
            jax.ShapeDtypeStruct((_N, 2 * _DH), jnp.float32),
        ],
    )(dis, agg, xst, wstk, avec, cvec)


def _stage4_body(dis_ref, a3_ref, e1st_ref, z01_ref, w2_ref, a2_ref, c2_ref,
                 wc_ref, bc_ref, out_ref):
    dis0 = dis_ref[:, 0:1]
    u = dis0 * jnp.concatenate(
        [a3_ref[0] + e1st_ref[0], a3_ref[1] + e1st_ref[1]], axis=1)
    e2 = jnp.maximum(_dot(u, w2_ref[...]) * a2_ref[0] + c2_ref[0], 0.0)
    h = jnp.concatenate([z01_ref[...], e2], axis=1)
    logits = _dot(h, wc_ref[...]) + bc_ref[0]
    m = jnp.max(logits, axis=-1, keepdims=True)
    ls = logits - m
    out_ref[...] = ls - jnp.log(jnp.sum(jnp.exp(ls), axis=-1, keepdims=True))


def _stage4(dis, a3, e1st, z01, w2, a2, c2, wc, bc):
    nb = _N // _BN
    return pl.pallas_call(
        _stage4_body,
        grid=(nb,),
        in_specs=[
            pl.BlockSpec((_BN, 3), lambda i: (i, 0)),
            pl.BlockSpec((2, _BN, _HD), lambda i: (0, i, 0)),
            pl.BlockSpec((2, _BN, _HD), lambda i: (0, i, 0)),
            pl.BlockSpec((_BN, 2 * _DH), lambda i: (i, 0)),
            pl.BlockSpec((_DH, _DH), lambda i: (0, 0)),
            pl.BlockSpec((1, _DH), lambda i: (0, 0)),
            pl.BlockSpec((1, _DH), lambda i: (0, 0)),
            pl.BlockSpec((3 * _DH, 64), lambda i: (0, 0)),
            pl.BlockSpec((1, 64), lambda i: (0, 0)),
        ],
        out_specs=pl.BlockSpec((_BN, 64), lambda i: (i, 0)),
        out_shape=jax.ShapeDtypeStruct((_N, 64), jnp.float32),
    )(dis, a3, e1st, z01, w2, a2, c2, wc, bc)


def kernel(x, adj, W_extra, b_extra, g_e, be_e, W_extra2, b_extra2, g_e2,
           be_e2, W0, b0, W1, b1, att, Wc, bc):
    f32 = jnp.float32
    src = adj[:, 0]
    dst = adj[:, 1]
    srcidx = (src.reshape(3, 1, _NS, _NCHA, _CH)
              + jnp.arange(_NC, dtype=adj.dtype).reshape(1, _NC, 1, 1, 1) * _N)
    dstidx = dst.reshape(3, _NS, _NCHA, _CH)
    dstdeg = dst.reshape(3, _NW, _NCHD, _CH)
    ones_deg = jnp.ones((_CH, _DEGW), f32)
    zeros_deg = jnp.zeros((_RPT, _DEGW), f32)
    zeros_feat = jnp.zeros((_RZ, _HD), f32)

    degp = _build_deg_kernel()(dstdeg, ones_deg, zeros_deg)
    dp2 = degp[:, :, :, :, 0].reshape(_NC, 3, _N).transpose(2, 0, 1).reshape(_N, 6)

    xst, dis = _prep(dp2, x)

    agg = _make_agg(3)(
        srcidx, dstidx, zeros_feat,
        xst[0].reshape(2 * _N, _HD),
        xst[1].reshape(2 * _N, _HD),
        xst[2].reshape(2 * _N, _HD))
    agg = agg.reshape(_NC, 3, _N, _HD)

    mask = jax.nn.softmax(att, axis=0)
    rs = 1.0 / float(np.sqrt(1.0 + 1e-5))
    scl = jnp.stack([jnp.float32(1.0), mask[0], mask[1]])
    avec = jnp.stack([g_e * rs, jnp.ones_like(b0), jnp.ones_like(b1)]) * scl[:, None]
    cvec = jnp.stack([g_e * b_extra * rs + be_e, b0, b1]) * scl[:, None]
    wstk = jnp.stack([W_extra, W0, W1])

    e1st, z01 = _stage2(dis, agg, xst, wstk, avec, cvec)

    a3 = _make_agg(1)(
        srcidx[0:1], dstidx[0:1], zeros_feat, e1st.reshape(2 * _N, _HD))
    a3 = a3.reshape(_NC, _N, _HD)

    a2 = (g_e2 * rs * mask[2]).reshape(1, _DH)
    c2 = ((g_e2 * b_extra2 * rs + be_e2) * mask[2]).reshape(1, _DH)
    z = _stage4(dis, a3, e1st, z01, W_extra2, a2, c2, Wc, bc.reshape(1, 64))
    return (z, jnp.zeros((), f32))


# speedup vs baseline: 1.1515x; 1.0231x over previous
import functools

import jax
import jax.numpy as jnp
import numpy as np
from jax import lax
from jax.experimental import pallas as pl
from jax.experimental.pallas import tpu as pltpu
from jax.experimental.pallas import tpu_sc as plsc

_N = 10000
_E = 320000
_DH = 128
_HD = 64
_NC = 2
_NS = 16
_NW = _NC * _NS
_CH = 125
_NCHD = (_E // _NW) // _CH
_NCHA = (_E // _NS) // _CH
_GRP = 5
_RPT = _N // _NS
_RZ = 125
_DEGW = 16
_BN = 2000


def _sc_mesh():
    return plsc.VectorSubcoreMesh(core_axis_name="c", subcore_axis_name="s")


@functools.cache
def _build_deg_kernel():
    return functools.partial(
        pl.kernel,
        out_type=jax.ShapeDtypeStruct((_NC, 3, _NS, _RPT, _DEGW), jnp.float32),
        mesh=_sc_mesh(),
        scratch_types=[
            pltpu.VMEM((_NCHD, _CH), jnp.int32),
            pltpu.VMEM((_CH, _DEGW), jnp.float32),
            pltpu.VMEM((_RPT, _DEGW), jnp.float32),
            pltpu.VMEM((_RZ, _DEGW), jnp.float32),
            pltpu.VMEM_SHARED((_N, _DEGW), jnp.float32),
            pltpu.SemaphoreType.DMA,
        ],
        compiler_params=pltpu.CompilerParams(use_tc_tiling_on_sc=False),
    )(_deg_body)


def _deg_body(dst_hbm, ones_hbm, zeros_hbm, out_hbm, dst_v, ones_v, zv, st, acc,
              dsem):
    cid = lax.axis_index("c")
    sid = lax.axis_index("s")
    wid = sid * _NC + cid
    pltpu.sync_copy(zeros_hbm, zv)
    pltpu.sync_copy(ones_hbm, ones_v)
    for k in range(3):
        pltpu.sync_copy(zv, acc.at[pl.ds(sid * _RPT, _RPT)])
        pltpu.sync_copy(dst_hbm.at[k, wid], dst_v)
        plsc.subcore_barrier()

        def fire(c, carry):
            pltpu.async_copy(ones_v, acc.at[dst_v.at[c]], dsem, add=True)
            return carry

        lax.fori_loop(0, _NCHD, fire, 0)

        def drain(c, carry):
            pltpu.make_async_copy(ones_v, acc.at[dst_v.at[c]], dsem).wait()
            return carry

        lax.fori_loop(0, _NCHD, drain, 0)
        plsc.subcore_barrier()
        for z in range(_RPT // _RZ):
            pltpu.sync_copy(acc.at[pl.ds(sid * _RPT + z * _RZ, _RZ)], st)
            pltpu.sync_copy(st, out_hbm.at[cid, k, sid, pl.ds(z * _RZ, _RZ)])


@functools.cache
def _make_agg(n):
    scratch = [
        pltpu.VMEM((_NCHA, _CH), jnp.int32),
        pltpu.VMEM((_NCHA, _CH), jnp.int32),
    ]
    scratch += [pltpu.VMEM((_CH, _HD), jnp.float32) for _ in range(_GRP)]
    scratch += [pltpu.VMEM_SHARED((_N, _HD), jnp.float32)]
    scratch += [pltpu.SemaphoreType.DMA for _ in range(2 * _GRP)]

    @functools.partial(
        pl.kernel,
        out_type=jax.ShapeDtypeStruct((_NC, n, _NS, _RPT, _HD), jnp.float32),
        mesh=_sc_mesh(),
        scratch_types=tuple(scratch),
        compiler_params=pltpu.CompilerParams(use_tc_tiling_on_sc=False),
    )
    def _agg(src_hbm, dst_hbm, zeros_hbm, *refs):
        tabs = refs[:n]
        out_hbm = refs[n]
        src_v, dst_v = refs[n + 1], refs[n + 2]
        bufs = refs[n + 3:n + 3 + _GRP]
        acc = refs[n + 3 + _GRP]
        gsem = refs[n + 4 + _GRP:n + 4 + 2 * _GRP]
        ssem = refs[n + 4 + 2 * _GRP:]
        cid = lax.axis_index("c")
        sid = lax.axis_index("s")
        for j in range(n):
            pltpu.sync_copy(zeros_hbm, bufs[0])
            for z in range(_RPT // _RZ):
                pltpu.sync_copy(bufs[0], acc.at[pl.ds(sid * _RPT + z * _RZ, _RZ)])
            pltpu.sync_copy(src_hbm.at[j, cid, sid], src_v)
            pltpu.sync_copy(dst_hbm.at[j, sid], dst_v)
            plsc.subcore_barrier()
            tab = tabs[j]

            for b in range(_GRP):
                pltpu.async_copy(tab.at[src_v.at[b]], bufs[b], gsem[b])

            def grp(i, carry, tab=tab):
                for b in range(_GRP):
                    c = i * _GRP + b
                    pltpu.make_async_copy(
                        tab.at[src_v.at[c]], bufs[b], gsem[b]).wait()
                    pltpu.async_copy(
                        bufs[b], acc.at[dst_v.at[c]], ssem[b], add=True)
                for b in range(_GRP):
                    c = i * _GRP + b
                    pltpu.make_async_copy(
                        bufs[b], acc.at[dst_v.at[c]], ssem[b]).wait()
                    pltpu.async_copy(
                        tab.at[src_v.at[c + _GRP]], bufs[b], gsem[b])
                return carry

            lax.fori_loop(0, _NCHA // _GRP - 1, grp, 0)
            base = _NCHA - _GRP
            for b in range(_GRP):
                pltpu.make_async_copy(
                    tab.at[src_v.at[base + b]], bufs[b], gsem[b]).wait()
                pltpu.async_copy(
                    bufs[b], acc.at[dst_v.at[base + b]], ssem[b], add=True)
            for b in range(_GRP):
                pltpu.make_async_copy(
                    bufs[b], acc.at[dst_v.at[base + b]], ssem[b]).wait()
            plsc.subcore_barrier()
            for z in range(_RPT // _RZ):
                pltpu.sync_copy(acc.at[pl.ds(sid * _RPT + z * _RZ, _RZ)], bufs[0])
                pltpu.sync_copy(bufs[0], out_hbm.at[cid, j, sid, pl.ds(z * _RZ, _RZ)])

    return _agg


def _prep_body(dp_ref, x_ref, xst_ref, dis_ref):
    dis = lax.rsqrt(dp_ref[:, :3] + dp_ref[:, 3:] + 1.0)
    dis_ref[...] = dis
    for k in range(3):
        sx = dis[:, k:k + 1] * x_ref[...]
        xst_ref[k, 0] = sx[:, :_HD]
        xst_ref[k, 1] = sx[:, _HD:]


def _prep(dp2, x):
    nb = _N // _BN
    return pl.pallas_call(
        _prep_body,
        grid=(nb,),
        in_specs=[
            pl.BlockSpec((_BN, 6), lambda i: (i, 0)),
            pl.BlockSpec((_BN, _DH), lambda i: (i, 0)),
        ],
        out_specs=[
            pl.BlockSpec((3, 2, _BN, _HD), lambda i: (0, 0, i, 0)),
            pl.BlockSpec((_BN, 3), lambda i: (i, 0)),
        ],
        out_shape=[
            jax.ShapeDtypeStruct((3, 2, _N, _HD), jnp.float32),
            jax.ShapeDtypeStruct((_N, 3), jnp.float32),
        ],
    )(dp2, x)


def _dot(a, b):
    return jnp.dot(a, b, preferred_element_type=jnp.float32,
                   precision=lax.Precision.HIGHEST)


def _stage2_body(dis_ref, agg_ref, xst_ref, w_ref, a_ref, c_ref,
                 e1st_ref, z01_ref):
    dis = dis_ref[...]
    u = [dis[:, k:k + 1] * jnp.concatenate(
            [agg_ref[0, k] + xst_ref[k, 0], agg_ref[1, k] + xst_ref[k, 1]],
            axis=1)
         for k in range(3)]
    e = jnp.maximum(_dot(u[0], w_ref[0]) * a_ref[0] + c_ref[0], 0.0)
    e1s = dis[:, 0:1] * e
    e1st_ref[0] = e1s[:, :_HD]
    e1st_ref[1] = e1s[:, _HD:]
    z0 = jnp.maximum(_dot(u[1], w_ref[1]) * a_ref[1] + c_ref[1], 0.0)
    z1 = jnp.maximum(_dot(u[2], w_ref[2]) * a_ref[2] + c_ref[2], 0.0)
    z01_ref[...] = jnp.concatenate([z0, z1], axis=1)


def _stage2(dis, agg, xst, wstk, avec, cvec):
    nb = _N // _BN
    return pl.pallas_call(
        _stage2_body,
        grid=(nb,),
        in_specs=[
            pl.BlockSpec((_BN, 3), lambda i: (i, 0)),
            pl.BlockSpec((2, 3, _BN, _HD), lambda i: (0, 0, i, 0)),
            pl.BlockSpec((3, 2, _BN, _HD), lambda i: (0, 0, i, 0)),
            pl.BlockSpec((3, _DH, _DH), lambda i: (0, 0, 0)),
            pl.BlockSpec((3, _DH), lambda i: (0, 0)),
            pl.BlockSpec((3, _DH), lambda i: (0, 0)),
        ],
        out_specs=[
            pl.BlockSpec((2, _BN, _HD), lambda i: (0, i, 0)),
            pl.BlockSpec((_BN, 2 * _DH), lambda i: (i, 0)),
        ],
        out_shape=[
            jax.ShapeDtypeStruct((2, _N, _HD), jnp.float32),
            jax.ShapeDtypeStruct((_N, 2 * _DH), jnp.float32),
        ],
    )(dis, agg, xst, wstk, avec, cvec)


def _stage4_body(dis_ref, a3_ref, e1st_ref, z01_ref, w2_ref, a2_ref, c2_ref,
                 wc_ref, bc_ref, out_ref):
    dis0 = dis_ref[:, 0:1]
    u = dis0 * jnp.concatenate(
        [a3_ref[0] + e1st_ref[0], a3_ref[1] + e1st_ref[1]], axis=1)
    e2 = jnp.maximum(_dot(u, w2_ref[...]) * a2_ref[0] + c2_ref[0], 0.0)
    h = jnp.concatenate([z01_ref[...], e2], axis=1)
    logits = _dot(h, wc_ref[...]) + bc_ref[0]
    m = jnp.max(logits, axis=-1, keepdims=True)
    ls = logits - m
    out_ref[...] = ls - jnp.log(jnp.sum(jnp.exp(ls), axis=-1, keepdims=True))


def _stage4(dis, a3, e1st, z01, w2, a2, c2, wc, bc):
    nb = _N // _BN
    return pl.pallas_call(
        _stage4_body,
        grid=(nb,),
        in_specs=[
            pl.BlockSpec((_BN, 3), lambda i: (i, 0)),
            pl.BlockSpec((2, _BN, _HD), lambda i: (0, i, 0)),
            pl.BlockSpec((2, _BN, _HD), lambda i: (0, i, 0)),
            pl.BlockSpec((_BN, 2 * _DH), lambda i: (i, 0)),
            pl.BlockSpec((_DH, _DH), lambda i: (0, 0)),
            pl.BlockSpec((1, _DH), lambda i: (0, 0)),
            pl.BlockSpec((1, _DH), lambda i: (0, 0)),
            pl.BlockSpec((3 * _DH, 64), lambda i: (0, 0)),
            pl.BlockSpec((1, 64), lambda i: (0, 0)),
        ],
        out_specs=pl.BlockSpec((_BN, 64), lambda i: (i, 0)),
        out_shape=jax.ShapeDtypeStruct((_N, 64), jnp.float32),
    )(dis, a3, e1st, z01, w2, a2, c2, wc, bc)


def kernel(x, adj, W_extra, b_extra, g_e, be_e, W_extra2, b_extra2, g_e2,
           be_e2, W0, b0, W1, b1, att, Wc, bc):
    f32 = jnp.float32
    src = adj[:, 0]
    dst = adj[:, 1]
    srcidx = (src.reshape(3, 1, _NS, _NCHA, _CH)
              + jnp.arange(_NC, dtype=adj.dtype).reshape(1, _NC, 1, 1, 1) * _N)
    dstidx = dst.reshape(3, _NS, _NCHA, _CH)
    dstdeg = dst.reshape(3, _NW, _NCHD, _CH)
    ones_deg = jnp.ones((_CH, _DEGW), f32)
    zeros_deg = jnp.zeros((_RPT, _DEGW), f32)
    zeros_feat = jnp.zeros((_RZ, _HD), f32)

    degp = _build_deg_kernel()(dstdeg, ones_deg, zeros_deg)
    dp2 = degp[:, :, :, :, 0].reshape(_NC, 3, _N).transpose(2, 0, 1).reshape(_N, 6)

    xst, dis = _prep(dp2, x)

    agg = _make_agg(3)(
        srcidx, dstidx, zeros_feat,
        xst[0].reshape(2 * _N, _HD),
        xst[1].reshape(2 * _N, _HD),
        xst[2].reshape(2 * _N, _HD))
    agg = agg.reshape(_NC, 3, _N, _HD)

    mask = jax.nn.softmax(att, axis=0)
    rs = 1.0 / float(np.sqrt(1.0 + 1e-5))
    scl = jnp.stack([jnp.float32(1.0), mask[0], mask[1]])
    avec = jnp.stack([g_e * rs, jnp.ones_like(b0), jnp.ones_like(b1)]) * scl[:, None]
    cvec = jnp.stack([g_e * b_extra * rs + be_e, b0, b1]) * scl[:, None]
    wstk = jnp.stack([W_extra, W0, W1])

    e1st, z01 = _stage2(dis, agg, xst, wstk, avec, cvec)

    a3 = _make_agg(1)(
        srcidx[0:1], dstidx[0:1], zeros_feat, e1st.reshape(2 * _N, _HD))
    a3 = a3.reshape(_NC, _N, _HD)

    a2 = (g_e2 * rs * mask[2]).reshape(1, _DH)
    c2 = ((g_e2 * b_extra2 * rs + be_e2) * mask[2]).reshape(1, _DH)
    z = _stage4(dis, a3, e1st, z01, W_extra2, a2, c2, Wc, bc.reshape(1, 64))
    return (z, jnp.zeros((), f32))
